# baseline (speedup 1.0000x reference)
import jax
import jax.numpy as jnp
from jax import lax
from jax.experimental import pallas as pl
from jax.experimental.pallas import tpu as pltpu
from jax.experimental.pallas import tpu_sc as plsc

NUM_ENT = 1000000
NUM_REL = 1000
EMB_DIM = 64
BATCH = 16384

NC = 2
NS = 16
L = 16
NW = NC * NS

B_PER_W = BATCH // NW
CHUNK = 128
N_CHUNKS = B_PER_W // CHUNK
GROUPS = CHUNK // L
NSEG = EMB_DIM // L


def _body(heads_hbm, rels_hbm, tails_hbm,
          ent_h_hbm, ent_t_hbm, rel_hbm, rel_inv_hbm,
          out_hbm,
          hidx, ridx, tidx,
          hh_v, ht_v, th_v, tt_v, r_v, rinv_v,
          tile16, out_v, sem):
    wid = lax.axis_index("s") * NC + lax.axis_index("c")
    base = wid * B_PER_W

    iota16 = lax.iota(jnp.int32, L)

    def chunk_body(c, _):
        cbase = base + c * CHUNK
        pltpu.sync_copy(heads_hbm.at[pl.ds(cbase, CHUNK)], hidx)
        pltpu.sync_copy(rels_hbm.at[pl.ds(cbase, CHUNK)], ridx)
        pltpu.sync_copy(tails_hbm.at[pl.ds(cbase, CHUNK)], tidx)
        cp1 = pltpu.make_async_copy(ent_h_hbm.at[hidx], hh_v, sem)
        cp2 = pltpu.make_async_copy(ent_h_hbm.at[tidx], ht_v, sem)
        cp3 = pltpu.make_async_copy(ent_t_hbm.at[hidx], th_v, sem)
        cp4 = pltpu.make_async_copy(ent_t_hbm.at[tidx], tt_v, sem)
        cp5 = pltpu.make_async_copy(rel_hbm.at[ridx], r_v, sem)
        cp6 = pltpu.make_async_copy(rel_inv_hbm.at[ridx], rinv_v, sem)
        for cp in (cp1, cp2, cp3, cp4, cp5, cp6):
            cp.start()
        for cp in (cp1, cp2, cp3, cp4, cp5, cp6):
            cp.wait()

        def group_body(g, _):
            eb = g * L
            for i in range(L):
                e = eb + i
                s = None
                for k in range(NSEG):
                    sl = pl.ds(k * L, L)
                    p = (hh_v[e, sl] * r_v[e, sl] * tt_v[e, sl]
                         + ht_v[e, sl] * rinv_v[e, sl] * th_v[e, sl])
                    s = p if s is None else s + p
                tile16[i, :] = s
            acc = jnp.zeros((L,), jnp.float32)
            for j in range(L):
                col = plsc.load_gather(
                    tile16, [iota16, jnp.full((L,), j, jnp.int32)])
                acc = acc + col
            score = jnp.clip(acc * 0.5, -20.0, 20.0)
            out_v[pl.ds(c * CHUNK + eb, L)] = score
            return ()

        lax.fori_loop(0, GROUPS, group_body, (), unroll=1)
        return ()

    lax.fori_loop(0, N_CHUNKS, chunk_body, (), unroll=1)
    pltpu.sync_copy(out_v, out_hbm.at[pl.ds(base, B_PER_W)])


@jax.jit
def kernel(heads, rels, tails, ent_h_embs, ent_t_embs, rel_embs,
           rel_inv_embs):
    mesh = plsc.VectorSubcoreMesh(core_axis_name="c", subcore_axis_name="s",
                                  num_cores=NC, num_subcores=NS)
    f = pl.kernel(
        _body,
        out_type=jax.ShapeDtypeStruct((BATCH,), jnp.float32),
        mesh=mesh,
        compiler_params=pltpu.CompilerParams(needs_layout_passes=False,
                                             use_tc_tiling_on_sc=False),
        scratch_types=[
            pltpu.VMEM((CHUNK,), jnp.int32),
            pltpu.VMEM((CHUNK,), jnp.int32),
            pltpu.VMEM((CHUNK,), jnp.int32),
            pltpu.VMEM((CHUNK, EMB_DIM), jnp.float32),
            pltpu.VMEM((CHUNK, EMB_DIM), jnp.float32),
            pltpu.VMEM((CHUNK, EMB_DIM), jnp.float32),
            pltpu.VMEM((CHUNK, EMB_DIM), jnp.float32),
            pltpu.VMEM((CHUNK, EMB_DIM), jnp.float32),
            pltpu.VMEM((CHUNK, EMB_DIM), jnp.float32),
            pltpu.VMEM((L, L), jnp.float32),
            pltpu.VMEM((B_PER_W,), jnp.float32),
            pltpu.SemaphoreType.DMA,
        ],
    )
    return f(heads.astype(jnp.int32), rels.astype(jnp.int32),
             tails.astype(jnp.int32), ent_h_embs, ent_t_embs,
             rel_embs, rel_inv_embs)


# speedup vs baseline: 1.2055x; 1.2055x over previous
import jax
import jax.numpy as jnp
from jax import lax
from jax.experimental import pallas as pl
from jax.experimental.pallas import tpu as pltpu
from jax.experimental.pallas import tpu_sc as plsc

NUM_ENT = 1000000
NUM_REL = 1000
EMB_DIM = 64
BATCH = 16384

NC = 2
NS = 16
L = 16
NW = NC * NS

B_PER_W = BATCH // NW
CHUNK = 128
N_CHUNKS = B_PER_W // CHUNK
GROUPS = CHUNK // L
NSEG = EMB_DIM // L


def _body(heads_hbm, rels_hbm, tails_hbm, comb_hbm, relcat_hbm,
          out_hbm,
          hidx, ridx, tidx,
          hrow_v, trow_v, rrow_v,
          tile16, out_v, sem):
    wid = lax.axis_index("s") * NC + lax.axis_index("c")
    base = wid * B_PER_W

    iota16 = lax.iota(jnp.int32, L)

    def chunk_body(c, _):
        cbase = base + c * CHUNK
        pltpu.sync_copy(heads_hbm.at[pl.ds(cbase, CHUNK)], hidx)
        pltpu.sync_copy(rels_hbm.at[pl.ds(cbase, CHUNK)], ridx)
        pltpu.sync_copy(tails_hbm.at[pl.ds(cbase, CHUNK)], tidx)
        cp1 = pltpu.make_async_copy(comb_hbm.at[hidx], hrow_v, sem)
        cp2 = pltpu.make_async_copy(comb_hbm.at[tidx], trow_v, sem)
        cp3 = pltpu.make_async_copy(relcat_hbm.at[ridx], rrow_v, sem)
        for cp in (cp1, cp2, cp3):
            cp.start()
        for cp in (cp1, cp2, cp3):
            cp.wait()

        def group_body(g, _):
            eb = g * L
            for i in range(L):
                e = eb + i
                s = None
                for k in range(NSEG):
                    lo = pl.ds(k * L, L)
                    hi = pl.ds(EMB_DIM + k * L, L)
                    p = (hrow_v[e, lo] * rrow_v[e, lo] * trow_v[e, hi]
                         + trow_v[e, lo] * rrow_v[e, hi] * hrow_v[e, hi])
                    s = p if s is None else s + p
                tile16[i, :] = s
            acc = jnp.zeros((L,), jnp.float32)
            for j in range(L):
                col = plsc.load_gather(
                    tile16, [iota16, jnp.full((L,), j, jnp.int32)])
                acc = acc + col
            score = jnp.clip(acc * 0.5, -20.0, 20.0)
            out_v[pl.ds(c * CHUNK + eb, L)] = score
            return ()

        lax.fori_loop(0, GROUPS, group_body, (), unroll=1)
        return ()

    lax.fori_loop(0, N_CHUNKS, chunk_body, (), unroll=1)
    pltpu.sync_copy(out_v, out_hbm.at[pl.ds(base, B_PER_W)])


@jax.jit
def kernel(heads, rels, tails, ent_h_embs, ent_t_embs, rel_embs,
           rel_inv_embs):
    comb = jnp.concatenate([ent_h_embs, ent_t_embs], axis=1)
    relcat = jnp.concatenate([rel_embs, rel_inv_embs], axis=1)
    mesh = plsc.VectorSubcoreMesh(core_axis_name="c", subcore_axis_name="s",
                                  num_cores=NC, num_subcores=NS)
    f = pl.kernel(
        _body,
        out_type=jax.ShapeDtypeStruct((BATCH,), jnp.float32),
        mesh=mesh,
        compiler_params=pltpu.CompilerParams(needs_layout_passes=False,
                                             use_tc_tiling_on_sc=False),
        scratch_types=[
            pltpu.VMEM((CHUNK,), jnp.int32),
            pltpu.VMEM((CHUNK,), jnp.int32),
            pltpu.VMEM((CHUNK,), jnp.int32),
            pltpu.VMEM((CHUNK, 2 * EMB_DIM), jnp.float32),
            pltpu.VMEM((CHUNK, 2 * EMB_DIM), jnp.float32),
            pltpu.VMEM((CHUNK, 2 * EMB_DIM), jnp.float32),
            pltpu.VMEM((L, L), jnp.float32),
            pltpu.VMEM((B_PER_W,), jnp.float32),
            pltpu.SemaphoreType.DMA,
        ],
    )
    return f(heads.astype(jnp.int32), rels.astype(jnp.int32),
             tails.astype(jnp.int32), comb, relcat)
